# baseline (speedup 1.0000x reference)
import functools

import jax
import jax.numpy as jnp
from jax import lax
from jax.experimental import pallas as pl
from jax.experimental.pallas import tpu as pltpu
from jax.experimental.pallas import tpu_sc as plsc

VOCAB_SIZE = 1000000
EMB = 64
BATCH = 16384
PAD = 50

NUM_CORES = 2
NUM_SUBCORES = 16
LANES = 16
NW = NUM_CORES * NUM_SUBCORES
NQ = EMB // LANES

SENT_PER_W = BATCH // NW
CH = 8
NCHUNK = SENT_PER_W // CH
IDX_COLS = 100
ROWS_PER_CHUNK = CH * PAD
GATHERS = ROWS_PER_CHUNK // IDX_COLS
IDX_ROWS_PER_W = SENT_PER_W * PAD // IDX_COLS

_mesh = plsc.VectorSubcoreMesh(core_axis_name="c", subcore_axis_name="s")


@functools.partial(
    pl.kernel,
    out_type=jax.ShapeDtypeStruct((BATCH, EMB), jnp.float32),
    mesh=_mesh,
    compiler_params=pltpu.CompilerParams(use_tc_tiling_on_sc=False),
    scratch_types=[
        pltpu.VMEM((2, GATHERS, IDX_COLS), jnp.int32),
        pltpu.VMEM((2, ROWS_PER_CHUNK, EMB), jnp.float32),
        pltpu.VMEM((2, CH, EMB), jnp.float32),
        pltpu.SemaphoreType.DMA((2,)),
        pltpu.SemaphoreType.DMA((2,)),
        pltpu.SemaphoreType.DMA((2,)),
    ],
)
def _bag(sent_hbm, table_hbm, out_hbm, idx_v, rows_v, out_v,
         gsem, osem, isem):
    wid = lax.axis_index("s") * NUM_CORES + lax.axis_index("c")
    irow_base = wid * IDX_ROWS_PER_W
    orow_base = wid * SENT_PER_W

    def load_idx(c, buf):
        pltpu.async_copy(
            sent_hbm.at[pl.ds(irow_base + c * GATHERS, GATHERS)],
            idx_v.at[buf], isem.at[buf])

    def fire_gathers(c, buf):
        pltpu.make_async_copy(
            sent_hbm.at[pl.ds(irow_base + c * GATHERS, GATHERS)],
            idx_v.at[buf], isem.at[buf]).wait()
        for m in range(GATHERS):
            pltpu.async_copy(
                table_hbm.at[idx_v.at[buf, m]],
                rows_v.at[buf].at[pl.ds(m * IDX_COLS, IDX_COLS)],
                gsem.at[buf])

    def process(c, buf):
        @pl.when(c >= 2)
        def _():
            pltpu.make_async_copy(
                out_v.at[buf], out_hbm.at[pl.ds(0, CH)],
                osem.at[buf]).wait()

        pltpu.make_async_copy(
            table_hbm.at[pl.ds(0, ROWS_PER_CHUNK)],
            rows_v.at[buf], gsem.at[buf]).wait()

        @pl.when(c + 2 < NCHUNK)
        def _():
            load_idx(c + 2, buf)

        rows = rows_v.at[buf]
        for s in range(CH):
            base = s * PAD
            acc = [rows[base, pl.ds(q * LANES, LANES)] for q in range(NQ)]
            for j in range(1, PAD):
                for q in range(NQ):
                    acc[q] = acc[q] + rows[base + j, pl.ds(q * LANES, LANES)]
            for q in range(NQ):
                out_v[buf, s, pl.ds(q * LANES, LANES)] = acc[q]

        pltpu.async_copy(
            out_v.at[buf], out_hbm.at[pl.ds(orow_base + c * CH, CH)],
            osem.at[buf])

        @pl.when(c + 2 < NCHUNK)
        def _():
            fire_gathers(c + 2, buf)

    load_idx(0, 0)
    fire_gathers(0, 0)
    load_idx(1, 1)
    fire_gathers(1, 1)

    def pair_body(k, carry):
        process(2 * k, 0)
        process(2 * k + 1, 1)
        return carry

    lax.fori_loop(0, NCHUNK // 2, pair_body, 0)

    for buf in range(2):
        pltpu.make_async_copy(
            out_v.at[buf], out_hbm.at[pl.ds(0, CH)], osem.at[buf]).wait()


def kernel(sentences, words_per_sentence, table):
    del words_per_sentence
    sent_rows = sentences.reshape(BATCH * PAD // IDX_COLS, IDX_COLS)
    return _bag(sent_rows, table)


# speedup vs baseline: 1.4157x; 1.4157x over previous
import functools

import jax
import jax.numpy as jnp
from jax import lax
from jax.experimental import pallas as pl
from jax.experimental.pallas import tpu as pltpu
from jax.experimental.pallas import tpu_sc as plsc

VOCAB_SIZE = 1000000
EMB = 64
BATCH = 16384
PAD = 50

NUM_CORES = 2
NUM_SUBCORES = 16
LANES = 16
NW = NUM_CORES * NUM_SUBCORES
NQ = EMB // LANES

SENT_PER_W = BATCH // NW
CH = 16
NCHUNK = SENT_PER_W // CH
IDX_COLS = 100
ROWS_PER_CHUNK = CH * PAD
GATHERS = ROWS_PER_CHUNK // IDX_COLS
IDX_ROWS_PER_W = SENT_PER_W * PAD // IDX_COLS

_mesh = plsc.VectorSubcoreMesh(core_axis_name="c", subcore_axis_name="s")


@functools.partial(
    pl.kernel,
    out_type=jax.ShapeDtypeStruct((BATCH, EMB), jnp.float32),
    mesh=_mesh,
    compiler_params=pltpu.CompilerParams(use_tc_tiling_on_sc=False),
    scratch_types=[
        pltpu.VMEM((2, GATHERS, IDX_COLS), jnp.int32),
        pltpu.VMEM((2, ROWS_PER_CHUNK, EMB), jnp.float32),
        pltpu.VMEM((2, CH, EMB), jnp.float32),
        pltpu.SemaphoreType.DMA((2,)),
        pltpu.SemaphoreType.DMA((2,)),
        pltpu.SemaphoreType.DMA((2,)),
    ],
)
def _bag(sent_hbm, table_hbm, out_hbm, idx_v, rows_v, out_v,
         gsem, osem, isem):
    wid = lax.axis_index("s") * NUM_CORES + lax.axis_index("c")
    irow_base = wid * IDX_ROWS_PER_W
    orow_base = wid * SENT_PER_W

    def load_idx(c, buf):
        pltpu.async_copy(
            sent_hbm.at[pl.ds(irow_base + c * GATHERS, GATHERS)],
            idx_v.at[buf], isem.at[buf])

    def fire_gathers(c, buf):
        pltpu.make_async_copy(
            sent_hbm.at[pl.ds(irow_base + c * GATHERS, GATHERS)],
            idx_v.at[buf], isem.at[buf]).wait()
        for m in range(GATHERS):
            pltpu.async_copy(
                table_hbm.at[idx_v.at[buf, m]],
                rows_v.at[buf].at[pl.ds(m * IDX_COLS, IDX_COLS)],
                gsem.at[buf])

    def process(c, buf):
        cur = lax.rem(c, 2) if buf is None else buf

        @pl.when(c >= 2)
        def _():
            pltpu.make_async_copy(
                out_v.at[cur], out_hbm.at[pl.ds(0, CH)],
                osem.at[cur]).wait()

        pltpu.make_async_copy(
            table_hbm.at[pl.ds(0, ROWS_PER_CHUNK)],
            rows_v.at[cur], gsem.at[cur]).wait()

        @pl.when(c + 2 < NCHUNK)
        def _():
            load_idx(c + 2, cur)

        def sent_body(s, carry2):
            base = s * PAD
            acc = [rows_v[cur, base, pl.ds(q * LANES, LANES)]
                   for q in range(NQ)]
            for j in range(1, PAD):
                for q in range(NQ):
                    acc[q] = acc[q] + rows_v[cur, base + j,
                                             pl.ds(q * LANES, LANES)]
            for q in range(NQ):
                out_v[cur, s, pl.ds(q * LANES, LANES)] = acc[q]
            return carry2

        lax.fori_loop(0, CH, sent_body, 0)

        pltpu.async_copy(
            out_v.at[cur], out_hbm.at[pl.ds(orow_base + c * CH, CH)],
            osem.at[cur])

        @pl.when(c + 2 < NCHUNK)
        def _():
            fire_gathers(c + 2, cur)

    load_idx(0, 0)
    fire_gathers(0, 0)
    load_idx(1, 1)
    fire_gathers(1, 1)

    def pair_body(k, carry):
        process(2 * k, 0)
        process(2 * k + 1, 1)
        return carry

    lax.fori_loop(0, NCHUNK // 2, pair_body, 0)

    for buf in range(2):
        pltpu.make_async_copy(
            out_v.at[buf], out_hbm.at[pl.ds(0, CH)], osem.at[buf]).wait()


def kernel(sentences, words_per_sentence, table):
    del words_per_sentence
    sent_rows = sentences.reshape(BATCH * PAD // IDX_COLS, IDX_COLS)
    return _bag(sent_rows, table)
